# baseline (speedup 1.0000x reference)
import jax
import jax.numpy as jnp
from jax.experimental import pallas as pl
from jax.experimental.pallas import tpu as pltpu

_B, _S, _D = 4, 8192, 768
_RANK = 4
_E = 16
_SCALE = 1.0 / _RANK

_TS = 4096
_NS = _S // _TS
_NO = 2
_HD = _D // _NO


def _fused_kernel(sid_ref, x_ref, Wt_ref, b_ref, A_ref, Bt_ref, out_ref, weff_ref):
    ss = pl.program_id(1)
    oo = pl.program_id(2)

    @pl.when((ss == 0) & (oo == 0))
    def _build_weff():
        weff_ref[...] = Wt_ref[...] + _SCALE * jnp.dot(
            A_ref[0].T, Bt_ref[0], preferred_element_type=jnp.float32
        )

    out_ref[0] = (
        jnp.dot(
            x_ref[0],
            weff_ref[:, pl.ds(oo * _HD, _HD)],
            preferred_element_type=jnp.float32,
        )
        + b_ref[...]
    )


def kernel(x, subject_id, W, b, lora_A, lora_B):
    Wt = W.T
    Bt = lora_B.transpose(0, 2, 1)
    sid = subject_id.astype(jnp.int32)

    grid_spec = pltpu.PrefetchScalarGridSpec(
        num_scalar_prefetch=1,
        grid=(_B, _NS, _NO),
        in_specs=[
            pl.BlockSpec((1, _TS, _D), lambda bb, ss, oo, sid_ref: (bb, ss, 0)),
            pl.BlockSpec((_D, _D), lambda bb, ss, oo, sid_ref: (0, 0)),
            pl.BlockSpec((1, _HD), lambda bb, ss, oo, sid_ref: (0, oo)),
            pl.BlockSpec(
                (1, _RANK, _D), lambda bb, ss, oo, sid_ref: (sid_ref[bb], 0, 0)
            ),
            pl.BlockSpec(
                (1, _RANK, _D), lambda bb, ss, oo, sid_ref: (sid_ref[bb], 0, 0)
            ),
        ],
        out_specs=pl.BlockSpec(
            (1, _TS, _HD), lambda bb, ss, oo, sid_ref: (bb, ss, oo)
        ),
        scratch_shapes=[pltpu.VMEM((_D, _D), jnp.float32)],
    )

    return pl.pallas_call(
        _fused_kernel,
        grid_spec=grid_spec,
        out_shape=jax.ShapeDtypeStruct((_B, _S, _D), jnp.float32),
        compiler_params=pltpu.CompilerParams(
            dimension_semantics=("arbitrary", "arbitrary", "arbitrary"),
            vmem_limit_bytes=100 * 1024 * 1024,
        ),
    )(sid, x, Wt, b.reshape(1, _D), lora_A, Bt)


# speedup vs baseline: 1.4722x; 1.4722x over previous
import jax
import jax.numpy as jnp
from jax.experimental import pallas as pl
from jax.experimental.pallas import tpu as pltpu

_B, _S, _D = 4, 8192, 768
_RANK = 4
_E = 16
_SCALE = 1.0 / _RANK

_TS = 4096


def _fused_kernel(sid_ref, x_ref, Wt_ref, b_ref, A_ref, Bt_ref, out_ref, weff_ref):
    @pl.when(pl.program_id(1) == 0)
    def _build_weff():
        weff_ref[...] = Wt_ref[...] + _SCALE * jnp.dot(
            A_ref[0].T, Bt_ref[0], preferred_element_type=jnp.float32
        )

    out_ref[0] = (
        jnp.dot(x_ref[0], weff_ref[...], preferred_element_type=jnp.float32)
        + b_ref[...]
    )


def kernel(x, subject_id, W, b, lora_A, lora_B):
    Wt = W.T
    Bt = lora_B.transpose(0, 2, 1)
    sid = subject_id.astype(jnp.int32)
    n_s = _S // _TS

    grid_spec = pltpu.PrefetchScalarGridSpec(
        num_scalar_prefetch=1,
        grid=(_B, n_s),
        in_specs=[
            pl.BlockSpec((1, _TS, _D), lambda bb, ss, sid_ref: (bb, ss, 0)),
            pl.BlockSpec((_D, _D), lambda bb, ss, sid_ref: (0, 0)),
            pl.BlockSpec((1, _D), lambda bb, ss, sid_ref: (0, 0)),
            pl.BlockSpec((1, _RANK, _D), lambda bb, ss, sid_ref: (sid_ref[bb], 0, 0)),
            pl.BlockSpec((1, _RANK, _D), lambda bb, ss, sid_ref: (sid_ref[bb], 0, 0)),
        ],
        out_specs=pl.BlockSpec((1, _TS, _D), lambda bb, ss, sid_ref: (bb, ss, 0)),
        scratch_shapes=[pltpu.VMEM((_D, _D), jnp.float32)],
    )

    return pl.pallas_call(
        _fused_kernel,
        grid_spec=grid_spec,
        out_shape=jax.ShapeDtypeStruct((_B, _S, _D), jnp.float32),
        compiler_params=pltpu.CompilerParams(
            dimension_semantics=("arbitrary", "arbitrary"),
            vmem_limit_bytes=100 * 1024 * 1024,
        ),
    )(sid, x, Wt, b.reshape(1, _D), lora_A, Bt)
